# baseline (speedup 1.0000x reference)
import functools
import jax
import jax.numpy as jnp
from jax import lax
from jax.experimental import pallas as pl
from jax.experimental.pallas import tpu as pltpu, tpu_sc as plsc

EMBED_DIM = 512
BATCH = 1024
_NUM_CORES = 2
_NUM_SUBCORES = 16
_NUM_WORKERS = _NUM_CORES * _NUM_SUBCORES
_B_PER_W = BATCH // _NUM_WORKERS
_HALF = _B_PER_W // 2

_mesh = plsc.VectorSubcoreMesh(core_axis_name="c", subcore_axis_name="s")

@functools.partial(
    pl.kernel,
    mesh=_mesh,
    out_type=jax.ShapeDtypeStruct((BATCH, EMBED_DIM), jnp.float32),
    compiler_params=pltpu.CompilerParams(use_tc_tiling_on_sc=False, skip_device_barrier=True),
    scratch_types=[
        pltpu.VMEM((_B_PER_W,), jnp.int32),
        pltpu.VMEM((_HALF, EMBED_DIM), jnp.float32),
        pltpu.VMEM((_HALF, EMBED_DIM), jnp.float32),
        pltpu.SemaphoreType.DMA,
        pltpu.SemaphoreType.DMA,
        pltpu.SemaphoreType.DMA,
        pltpu.SemaphoreType.DMA,
    ],
)
def _gather_rows(table_hbm, idx_hbm, out_hbm, idx_v, rows0, rows1, g0, g1, s0, s1):
    wid = lax.axis_index("s") * _NUM_CORES + lax.axis_index("c")
    base = wid * _B_PER_W
    pltpu.sync_copy(idx_hbm.at[pl.ds(base, _B_PER_W)], idx_v)
    c0 = pltpu.async_copy(table_hbm.at[idx_v.at[pl.ds(0, _HALF)]], rows0, g0)
    c1 = pltpu.async_copy(table_hbm.at[idx_v.at[pl.ds(_HALF, _HALF)]], rows1, g1)
    c0.wait()
    w0 = pltpu.async_copy(rows0, out_hbm.at[pl.ds(base, _HALF)], s0)
    c1.wait()
    w1 = pltpu.async_copy(rows1, out_hbm.at[pl.ds(base + _HALF, _HALF)], s1)
    w0.wait()
    w1.wait()

def kernel(x, t, embeddings):
    return _gather_rows(embeddings, t.astype(jnp.int32))[:, :, None, None]


# speedup vs baseline: 1.0148x; 1.0148x over previous
import functools
import jax
import jax.numpy as jnp
from jax import lax
from jax.experimental import pallas as pl
from jax.experimental.pallas import tpu as pltpu, tpu_sc as plsc

EMBED_DIM = 512
BATCH = 1024
_NUM_SUBCORES = 16
_B_PER_W = BATCH // _NUM_SUBCORES

_mesh = plsc.VectorSubcoreMesh(core_axis_name="c", subcore_axis_name="s", num_cores=1)

@functools.partial(
    pl.kernel,
    mesh=_mesh,
    out_type=jax.ShapeDtypeStruct((BATCH, EMBED_DIM), jnp.float32),
    compiler_params=pltpu.CompilerParams(use_tc_tiling_on_sc=False),
    scratch_types=[
        pltpu.VMEM((_B_PER_W,), jnp.int32),
        pltpu.VMEM((_B_PER_W, EMBED_DIM), jnp.float32),
        pltpu.SemaphoreType.DMA,
    ],
)
def _gather_rows(table_hbm, idx_hbm, out_hbm, idx_v, rows_v, sem):
    base = lax.axis_index("s") * _B_PER_W
    pltpu.sync_copy(idx_hbm.at[pl.ds(base, _B_PER_W)], idx_v)
    pltpu.async_copy(table_hbm.at[idx_v], rows_v, sem).wait()
    pltpu.sync_copy(rows_v, out_hbm.at[pl.ds(base, _B_PER_W)])

def kernel(x, t, embeddings):
    return _gather_rows(embeddings, t.astype(jnp.int32))[:, :, None, None]


# speedup vs baseline: 1.0222x; 1.0073x over previous
import functools

import jax
import jax.numpy as jnp
from jax import lax
from jax.experimental import pallas as pl
from jax.experimental.pallas import tpu as pltpu, tpu_sc as plsc

TIME_STEPS = 1000
EMBED_DIM = 512
BATCH = 1024
_LANE = 128
_CHUNKS = EMBED_DIM // _LANE

_NUM_CORES = 2
_NUM_SUBCORES = 16
_NUM_WORKERS = _NUM_CORES * _NUM_SUBCORES
_B_PER_W = BATCH // _NUM_WORKERS
_C_PER_W = _B_PER_W * _CHUNKS

_mesh = plsc.VectorSubcoreMesh(core_axis_name="c", subcore_axis_name="s")


@functools.partial(
    pl.kernel,
    mesh=_mesh,
    out_type=jax.ShapeDtypeStruct((BATCH * _CHUNKS, _LANE), jnp.float32),
    compiler_params=pltpu.CompilerParams(use_tc_tiling_on_sc=False, needs_layout_passes=False),
    scratch_types=[
        pltpu.VMEM((_B_PER_W,), jnp.int32),
        pltpu.VMEM((_C_PER_W,), jnp.int32),
        pltpu.VMEM((_C_PER_W, _LANE), jnp.float32),
        pltpu.SemaphoreType.DMA,
    ],
)
def _gather_rows(table_hbm, idx_hbm, out_hbm, idx_v, idx4_v, rows_v, sem):
    wid = lax.axis_index("s") * _NUM_CORES + lax.axis_index("c")
    base = wid * _B_PER_W
    pltpu.sync_copy(idx_hbm.at[pl.ds(base, _B_PER_W)], idx_v)
    for k in range(_B_PER_W // 16):
        tv = idx_v[pl.ds(k * 16, 16)]
        pb = ((tv >> 3) << 5) | (tv & 7)
        pos = lax.iota(jnp.int32, 16) * _CHUNKS + k * 16 * _CHUNKS
        for c in range(_CHUNKS):
            plsc.store_scatter(idx4_v, [pos + c], pb + c * 8)
    pltpu.async_copy(table_hbm.at[idx4_v], rows_v, sem).wait()
    pltpu.sync_copy(rows_v, out_hbm.at[pl.ds(wid * _C_PER_W, _C_PER_W)])


def kernel(x, t, embeddings):
    table4 = (
        embeddings.reshape(TIME_STEPS // 8, 8, _CHUNKS, _LANE)
        .transpose(0, 2, 1, 3)
        .reshape(TIME_STEPS * _CHUNKS, _LANE)
    )
    out = _gather_rows(table4, t.astype(jnp.int32))
    return out.reshape(BATCH, EMBED_DIM, 1, 1)


# speedup vs baseline: 1.0447x; 1.0220x over previous
import functools
import jax
import jax.numpy as jnp
from jax import lax
from jax.experimental import pallas as pl
from jax.experimental.pallas import tpu as pltpu, tpu_sc as plsc

TIME_STEPS = 1000
EMBED_DIM = 512
BATCH = 1024
_LANE = 128
_CHUNKS = EMBED_DIM // _LANE

_NUM_SUBCORES = 16
_B_PER_W = BATCH // _NUM_SUBCORES
_BLK = 32
_C_BLK = _BLK * _CHUNKS

_mesh = plsc.VectorSubcoreMesh(core_axis_name="c", subcore_axis_name="s", num_cores=1)


@functools.partial(
    pl.kernel,
    mesh=_mesh,
    out_type=jax.ShapeDtypeStruct((BATCH * _CHUNKS, _LANE), jnp.float32),
    compiler_params=pltpu.CompilerParams(use_tc_tiling_on_sc=False, needs_layout_passes=False),
    scratch_types=[
        pltpu.VMEM((_B_PER_W,), jnp.int32),
        pltpu.VMEM((_C_BLK,), jnp.int32),
        pltpu.VMEM((_C_BLK,), jnp.int32),
        pltpu.VMEM((_C_BLK, _LANE), jnp.float32),
        pltpu.VMEM((_C_BLK, _LANE), jnp.float32),
        pltpu.SemaphoreType.DMA,
        pltpu.SemaphoreType.DMA,
        pltpu.SemaphoreType.DMA,
        pltpu.SemaphoreType.DMA,
    ],
)
def _gather_rows(table_hbm, idx_hbm, out_hbm, idx_v, idxA, idxB, rowsA, rowsB,
                 gA, gB, sA, sB):
    sid = lax.axis_index("s")
    base = sid * _B_PER_W
    pltpu.sync_copy(idx_hbm.at[pl.ds(base, _B_PER_W)], idx_v)

    def fill(idx4, off):
        for k in range(_BLK // 16):
            tv = idx_v[pl.ds(off + k * 16, 16)]
            pb = ((tv >> 3) << 5) | (tv & 7)
            pos = lax.iota(jnp.int32, 16) * _CHUNKS + k * 16 * _CHUNKS
            for c in range(_CHUNKS):
                plsc.store_scatter(idx4, [pos + c], pb + c * 8)

    fill(idxA, 0)
    cA = pltpu.async_copy(table_hbm.at[idxA], rowsA, gA)
    fill(idxB, _BLK)
    cB = pltpu.async_copy(table_hbm.at[idxB], rowsB, gB)
    cA.wait()
    wA = pltpu.async_copy(rowsA, out_hbm.at[pl.ds(base * _CHUNKS, _C_BLK)], sA)
    cB.wait()
    wB = pltpu.async_copy(rowsB, out_hbm.at[pl.ds(base * _CHUNKS + _C_BLK, _C_BLK)], sB)
    wA.wait()
    wB.wait()


def kernel(x, t, embeddings):
    table4 = (
        embeddings.reshape(TIME_STEPS // 8, 8, _CHUNKS, _LANE)
        .transpose(0, 2, 1, 3)
        .reshape(TIME_STEPS * _CHUNKS, _LANE)
    )
    out = _gather_rows(table4, t.astype(jnp.int32))
    return out.reshape(BATCH, EMBED_DIM, 1, 1)
